# Initial kernel scaffold; baseline (speedup 1.0000x reference)
import jax
import jax.numpy as jnp
from jax.experimental import pallas as pl


def kernel(x, rois):
    raise NotImplementedError("write your pallas kernel here")


# speedup vs baseline: 9.5602x; 9.5602x over previous
import jax
import jax.numpy as jnp
from jax.experimental import pallas as pl
from jax.experimental.pallas import tpu as pltpu

OUTH = 7
OUTW = 7
SCALE = 0.0625
B, C, H, W = 2, 512, 50, 50
N = 300
KH = 10
KW = 10
NEG = -3.0e38


def _body(bidx_s, hs_s, he_s, ws_s, we_s, x_ref, out_ref):
    n = pl.program_id(0)
    b = bidx_s[n]

    h_iota = jax.lax.broadcasted_iota(jnp.int32, (KH, 1, 1), 0)
    w_iota = jax.lax.broadcasted_iota(jnp.int32, (W, 1), 0)

    for ph in range(OUTH):
        hs = hs_s[n, ph]
        he = he_s[n, ph]
        hs_c = jnp.clip(hs, 0, H - KH)
        band = x_ref[b, pl.ds(hs_c, KH)]
        h_idx = hs_c + h_iota
        mh = (h_idx >= hs) & (h_idx < he)
        band = jnp.where(mh, band, NEG)
        rowmax = jnp.max(band, axis=0)
        for pw in range(OUTW):
            ws = ws_s[n, pw]
            we = we_s[n, pw]
            mw = (w_iota >= ws) & (w_iota < we)
            seg = jnp.where(mw, rowmax, NEG)
            mx = jnp.max(seg, axis=0)
            valid = (he > hs) & (we > ws)
            mx = jnp.where(valid, mx, 0.0)
            out_ref[0, ph * OUTW + pw] = mx


def _bin_bounds(rois):
    bidx = rois[:, 0].astype(jnp.int32)
    xmin = jnp.round(rois[:, 1] * SCALE).astype(jnp.int32)
    ymin = jnp.round(rois[:, 2] * SCALE).astype(jnp.int32)
    xmax = jnp.round(rois[:, 3] * SCALE).astype(jnp.int32)
    ymax = jnp.round(rois[:, 4] * SCALE).astype(jnp.int32)
    roi_w = jnp.maximum(xmax - xmin + 1, 1).astype(jnp.float32)
    roi_h = jnp.maximum(ymax - ymin + 1, 1).astype(jnp.float32)
    bin_h = roi_h / OUTH
    bin_w = roi_w / OUTW
    hs = jnp.stack([jnp.clip(jnp.floor(ph * bin_h).astype(jnp.int32) + ymin, 0, H)
                    for ph in range(OUTH)], axis=1)
    he = jnp.stack([jnp.clip(jnp.ceil((ph + 1) * bin_h).astype(jnp.int32) + ymin, 0, H)
                    for ph in range(OUTH)], axis=1)
    ws = jnp.stack([jnp.clip(jnp.floor(pw * bin_w).astype(jnp.int32) + xmin, 0, W)
                    for pw in range(OUTW)], axis=1)
    we = jnp.stack([jnp.clip(jnp.ceil((pw + 1) * bin_w).astype(jnp.int32) + xmin, 0, W)
                    for pw in range(OUTW)], axis=1)
    return bidx, hs, he, ws, we


def kernel(x, rois):
    xt = jnp.transpose(x, (0, 2, 3, 1))
    bidx, hs, he, ws, we = _bin_bounds(rois)
    out = pl.pallas_call(
        _body,
        grid_spec=pltpu.PrefetchScalarGridSpec(
            num_scalar_prefetch=5,
            grid=(N,),
            in_specs=[
                pl.BlockSpec((B, H, W, C), lambda n, *refs: (0, 0, 0, 0)),
            ],
            out_specs=pl.BlockSpec((1, OUTH * OUTW, C), lambda n, *refs: (n, 0, 0)),
        ),
        out_shape=jax.ShapeDtypeStruct((N, OUTH * OUTW, C), jnp.float32),
    )(bidx, hs, he, ws, we, xt)
    return jnp.transpose(out, (0, 2, 1)).reshape(N, C, OUTH, OUTW)
